# baseline (speedup 1.0000x reference)
import functools

import jax
import jax.numpy as jnp
from jax import lax
from jax.experimental import pallas as pl
from jax.experimental.pallas import tpu as pltpu
from jax.experimental.pallas import tpu_sc as plsc

DIM = 64
NC = 2
NS = 16
NW = NC * NS
R = 4


def _emb_body(table_hbm, x_hbm, out_hbm,
              idx0, idx1, rows0, rows1, gsem0, gsem1, osem0, osem1):
    wid = lax.axis_index("s") * NC + lax.axis_index("c")
    S, T = x_hbm.shape
    rows_per_w = S // NW
    n_chunks = rows_per_w // R
    wbase = wid * rows_per_w

    idx_v = (idx0, idx1)
    rows_v = (rows0, rows1)
    gsem = (gsem0, gsem1)
    osem = (osem0, osem1)

    def fire_gathers(slot, c):
        pltpu.sync_copy(x_hbm.at[pl.ds(wbase + c * R, R)], idx_v[slot])
        for k in range(R):
            pltpu.async_copy(table_hbm.at[idx_v[slot].at[k]],
                             rows_v[slot].at[k], gsem[slot])

    def wait_gathers(slot):
        for k in range(R):
            pltpu.make_async_copy(table_hbm.at[idx_v[slot].at[k]],
                                  rows_v[slot].at[k], gsem[slot]).wait()

    def out_slice(c):
        return out_hbm.at[pl.ds(wbase + c * R, R)]

    fire_gathers(0, 0)

    def outer(j, carry):
        for t in (0, 1):
            c = 2 * j + t
            nt = 1 - t

            @pl.when(c + 1 < n_chunks)
            def _fire_next():
                @pl.when(c >= 1)
                def _drain_prev_write():
                    pltpu.make_async_copy(
                        rows_v[nt], out_slice(c - 1), osem[nt]).wait()
                fire_gathers(nt, c + 1)

            wait_gathers(t)
            pltpu.async_copy(rows_v[t], out_slice(c), osem[t])
        return carry

    lax.fori_loop(0, n_chunks // 2, outer, 0)

    pltpu.make_async_copy(rows_v[0], out_slice(n_chunks - 2), osem[0]).wait()
    pltpu.make_async_copy(rows_v[1], out_slice(n_chunks - 1), osem[1]).wait()


@jax.jit
def kernel(X, table):
    S, T = X.shape
    mesh = plsc.VectorSubcoreMesh(core_axis_name="c", subcore_axis_name="s")
    k = functools.partial(
        pl.kernel,
        mesh=mesh,
        out_type=jax.ShapeDtypeStruct((S, T, DIM), jnp.float32),
        scratch_types=[
            pltpu.VMEM((R, 200), jnp.int32),
            pltpu.VMEM((R, 200), jnp.int32),
            pltpu.VMEM((R, 200, DIM), jnp.float32),
            pltpu.VMEM((R, 200, DIM), jnp.float32),
            pltpu.SemaphoreType.DMA,
            pltpu.SemaphoreType.DMA,
            pltpu.SemaphoreType.DMA,
            pltpu.SemaphoreType.DMA,
        ],
        compiler_params=pltpu.CompilerParams(use_tc_tiling_on_sc=False),
    )(_emb_body)
    return k(table, X.astype(jnp.int32))
